# Initial kernel scaffold; baseline (speedup 1.0000x reference)
import jax
import jax.numpy as jnp
from jax.experimental import pallas as pl


def kernel(input, out_in):
    raise NotImplementedError("write your pallas kernel here")


# speedup vs baseline: 8.8527x; 8.8527x over previous
import functools

import jax
import jax.numpy as jnp
from jax.experimental import pallas as pl
from jax.experimental.pallas import tpu as pltpu

N = 8192
T = 32
K = 409


def _activation_kernel(x_ref, noise_ref, o_ref, *, is_in):
    x = x_ref[...]
    xmax = jnp.max(x, axis=1, keepdims=True)
    xmin = jnp.min(x, axis=1, keepdims=True)
    if is_in:
        scale = (1e-10 + xmax - xmin) / 10.0
    else:
        scale = jnp.abs(xmin / 10.0)
    y = x + scale * noise_ref[...]

    u = jax.lax.bitcast_convert_type(y, jnp.uint32)
    neg = u >= jnp.uint32(0x80000000)
    s = jnp.where(neg, ~u, u | jnp.uint32(0x80000000))

    v = jnp.zeros((x.shape[0], 1), jnp.uint32)
    for bit in range(31, -1, -1):
        cand = v | jnp.uint32(1 << bit)
        cnt = jnp.sum((s >= cand).astype(jnp.int32), axis=1, keepdims=True)
        v = jnp.where(cnt >= K, cand, v)

    cnt_gt = jnp.sum((s > v).astype(jnp.int32), axis=1, keepdims=True)
    need = K - cnt_gt
    idx = jax.lax.broadcasted_iota(jnp.int32, x.shape, 1)
    tie = s == v
    b = jnp.zeros((x.shape[0], 1), jnp.int32)
    for bit in range(12, -1, -1):
        cand = b | (1 << bit)
        f = jnp.sum((tie & (idx < cand)).astype(jnp.int32), axis=1, keepdims=True)
        b = jnp.where(f < need, cand, b)

    mask = (s > v) | (tie & (idx <= b))
    o_ref[...] = mask.astype(jnp.float32)


def _activation(x, noise, is_in):
    return pl.pallas_call(
        functools.partial(_activation_kernel, is_in=is_in),
        out_shape=jax.ShapeDtypeStruct((T, N), jnp.float32),
    )(x, noise)


def _matmul_kernel(a_ref, b_ref, o_ref):
    k = pl.program_id(1)

    @pl.when(k == 0)
    def _():
        o_ref[...] = jnp.zeros_like(o_ref)

    o_ref[...] += jax.lax.dot_general(
        a_ref[...], b_ref[...],
        (((1,), (1,)), ((), ())),
        preferred_element_type=jnp.float32,
    )


def _matmul(a, w):
    bo, bk = 512, 2048
    return pl.pallas_call(
        _matmul_kernel,
        grid=(N // bo, N // bk),
        in_specs=[
            pl.BlockSpec((T, bk), lambda j, k: (0, k)),
            pl.BlockSpec((bo, bk), lambda j, k: (j, k)),
        ],
        out_specs=pl.BlockSpec((T, bo), lambda j, k: (0, j)),
        out_shape=jax.ShapeDtypeStruct((T, N), jnp.float32),
        compiler_params=pltpu.CompilerParams(
            dimension_semantics=("parallel", "arbitrary"),
        ),
    )(a, w)


def kernel(input, out_in):
    base = jax.random.key(42)
    noise_in = jnp.stack(
        [jax.random.normal(jax.random.fold_in(base, 2 * t), (N,), jnp.float32)
         for t in range(T)])
    noise_out = jnp.stack(
        [jax.random.normal(jax.random.fold_in(base, 2 * t + 1), (N,), jnp.float32)
         for t in range(T)])

    in_ = _activation(input, noise_in, is_in=True)
    out_hat = _matmul(in_, out_in)
    return _activation(out_hat, noise_out, is_in=False)


# speedup vs baseline: 24.0360x; 2.7151x over previous
import functools

import jax
import jax.numpy as jnp
from jax.experimental import pallas as pl
from jax.experimental.pallas import tpu as pltpu

N = 8192
T = 32
K = 409


def _activation_kernel(x_ref, noise_ref, o_ref, *, is_in):
    x = x_ref[...]
    xmax = jnp.max(x, axis=1, keepdims=True)
    xmin = jnp.min(x, axis=1, keepdims=True)
    if is_in:
        scale = (1e-10 + xmax - xmin) / 10.0
    else:
        scale = jnp.abs(xmin / 10.0)
    y = x + scale * noise_ref[...]

    u = jax.lax.bitcast_convert_type(y, jnp.uint32)
    neg = u >= jnp.uint32(0x80000000)
    s = jnp.where(neg, ~u, u | jnp.uint32(0x80000000))

    v = jnp.zeros((x.shape[0], 1), jnp.uint32)
    for bit in range(31, -1, -1):
        cand = v | jnp.uint32(1 << bit)
        cnt = jnp.sum((s >= cand).astype(jnp.int32), axis=1, keepdims=True)
        v = jnp.where(cnt >= K, cand, v)

    cnt_gt = jnp.sum((s > v).astype(jnp.int32), axis=1, keepdims=True)
    need = K - cnt_gt
    idx = jax.lax.broadcasted_iota(jnp.int32, x.shape, 1)
    tie = s == v
    b = jnp.zeros((x.shape[0], 1), jnp.int32)
    for bit in range(12, -1, -1):
        cand = b | (1 << bit)
        f = jnp.sum((tie & (idx < cand)).astype(jnp.int32), axis=1, keepdims=True)
        b = jnp.where(f < need, cand, b)

    mask = (s > v) | (tie & (idx <= b))
    o_ref[...] = mask.astype(jnp.float32)


def _activation(x, noise, is_in):
    return pl.pallas_call(
        functools.partial(_activation_kernel, is_in=is_in),
        out_shape=jax.ShapeDtypeStruct((T, N), jnp.float32),
    )(x, noise)


def _matmul_kernel(a_ref, b_ref, o_ref):
    k = pl.program_id(1)

    @pl.when(k == 0)
    def _():
        o_ref[...] = jnp.zeros_like(o_ref)

    o_ref[...] += jax.lax.dot_general(
        a_ref[...], b_ref[...],
        (((1,), (1,)), ((), ())),
        preferred_element_type=jnp.float32,
    )


def _matmul(a, w):
    bo, bk = 512, 2048
    return pl.pallas_call(
        _matmul_kernel,
        grid=(N // bo, N // bk),
        in_specs=[
            pl.BlockSpec((T, bk), lambda j, k: (0, k)),
            pl.BlockSpec((bo, bk), lambda j, k: (j, k)),
        ],
        out_specs=pl.BlockSpec((T, bo), lambda j, k: (0, j)),
        out_shape=jax.ShapeDtypeStruct((T, N), jnp.float32),
        compiler_params=pltpu.CompilerParams(
            dimension_semantics=("parallel", "arbitrary"),
        ),
    )(a, w)


def kernel(input, out_in):
    base = jax.random.key(42)
    keys = jax.vmap(jax.random.fold_in, in_axes=(None, 0))(base, jnp.arange(2 * T))
    noise = jax.vmap(lambda k: jax.random.normal(k, (N,), jnp.float32))(keys)
    noise_in = noise[0::2]
    noise_out = noise[1::2]

    in_ = _activation(input, noise_in, is_in=True)
    out_hat = _matmul(in_, out_in)
    return _activation(out_hat, noise_out, is_in=False)
